# Initial kernel scaffold; baseline (speedup 1.0000x reference)
import jax
import jax.numpy as jnp
from jax.experimental import pallas as pl


def kernel(x, W, scale, per_expert_scale):
    raise NotImplementedError("write your pallas kernel here")


# speedup vs baseline: 1.3120x; 1.3120x over previous
import functools

import jax
import jax.numpy as jnp
from jax import lax
from jax.experimental import pallas as pl
from jax.experimental.pallas import tpu as pltpu
from jax.experimental.pallas import tpu_sc as plsc

HIDDEN = 4096
NEXP = 64
TOPK = 2
NTOK = 16384
EPS = 1e-06

BT1 = 512
NB1 = NTOK // BT1
BT2 = 512
NB2 = NTOK // BT2
NW = 32
ITEMS = NTOK * TOPK
IPW = ITEMS // NW
IDXROWS = IPW // 128


def _router_body(x_ref, w_ref, scale_ref, pes_ref, keys_ref, vals_ref, hist_ref):
    xb = x_ref[...]
    ms = jnp.mean(xb * xb, axis=1, keepdims=True)
    normed = xb * lax.rsqrt(ms + EPS) * scale_ref[...]
    logits = lax.dot_general(normed, w_ref[...], (((1,), (1,)), ((), ())))
    logits = logits * pes_ref[...]
    scores = jax.nn.sigmoid(logits)
    iot = lax.broadcasted_iota(jnp.int32, scores.shape, 1)
    m1 = jnp.max(scores, axis=1, keepdims=True)
    i1 = jnp.min(jnp.where(scores == m1, iot, NEXP), axis=1, keepdims=True)
    sc2 = jnp.where(iot == i1, -1.0, scores)
    m2 = jnp.max(sc2, axis=1, keepdims=True)
    i2 = jnp.min(jnp.where(sc2 == m2, iot, NEXP), axis=1, keepdims=True)
    keys_ref[...] = jnp.concatenate([i1, i2], axis=1)
    vals_ref[...] = jnp.concatenate([m1, m2], axis=1)
    oh = (iot == i1).astype(jnp.int32) + (iot == i2).astype(jnp.int32)
    hist_ref[...] = jnp.sum(oh, axis=0).reshape(1, 1, NEXP)


def _pos_body(keys_ref, hist_ref, pos_ref, cnt_ref, run_ref):
    b = pl.program_id(0)
    total = jnp.sum(hist_ref[...], axis=(0, 1)).reshape(1, NEXP)
    totf = total.astype(jnp.float32)
    ir = lax.broadcasted_iota(jnp.int32, (NEXP, NEXP), 0)
    ic = lax.broadcasted_iota(jnp.int32, (NEXP, NEXP), 1)
    lmask = (ic < ir).astype(jnp.float32)
    offs = lax.dot_general(totf, lmask, (((1,), (1,)), ((), ())),
                           precision=lax.Precision.HIGHEST)

    @pl.when(b == 0)
    def _():
        run_ref[...] = offs

    run = run_ref[...]
    keys = keys_ref[...]
    k0 = keys[:, 0:1]
    k1 = keys[:, 1:2]
    iot = lax.broadcasted_iota(jnp.int32, (BT2, NEXP), 1)
    oh0 = (iot == k0).astype(jnp.float32)
    oh1 = (iot == k1).astype(jnp.float32)
    ohs = oh0 + oh1
    irb = lax.broadcasted_iota(jnp.int32, (BT2, BT2), 0)
    icb = lax.broadcasted_iota(jnp.int32, (BT2, BT2), 1)
    ltri = (icb < irb).astype(jnp.float32)
    prefix = lax.dot_general(ltri, ohs, (((1,), (0,)), ((), ())),
                             precision=lax.Precision.HIGHEST)
    base = prefix + run
    p0 = jnp.sum(oh0 * base, axis=1, keepdims=True)
    p1 = jnp.sum(oh1 * base, axis=1, keepdims=True)
    pos_ref[...] = jnp.concatenate([p0, p1], axis=1).astype(jnp.int32)
    run_ref[...] = run + jnp.sum(ohs, axis=0, keepdims=True)
    cnt_ref[...] = total


@functools.cache
def _make_sc_scatter():
    mesh = plsc.VectorSubcoreMesh(core_axis_name="c", subcore_axis_name="s")

    @functools.partial(
        pl.kernel,
        mesh=mesh,
        out_type=[
            jax.ShapeDtypeStruct((ITEMS,), jnp.float32),
            jax.ShapeDtypeStruct((ITEMS,), jnp.int32),
        ],
        scratch_types=[
            pltpu.VMEM((IDXROWS, 128), jnp.int32),
            pltpu.VMEM((IDXROWS, 128), jnp.float32),
            pltpu.VMEM((IDXROWS, 128), jnp.int32),
            pltpu.SemaphoreType.DMA,
            pltpu.SemaphoreType.DMA,
        ],
    )
    def _sc_scatter(pos_hbm, val_hbm, tok_hbm, outs_hbm, outt_hbm,
                    pos_v, val_v, tok_v, sem_a, sem_b):
        wid = lax.axis_index("s") * 2 + lax.axis_index("c")
        pltpu.sync_copy(pos_hbm.at[wid], pos_v)
        pltpu.sync_copy(val_hbm.at[wid], val_v)
        pltpu.sync_copy(tok_hbm.at[wid], tok_v)
        handles = []
        for j in range(IDXROWS):
            handles.append(
                pltpu.async_copy(val_v.at[j], outs_hbm.at[pos_v.at[j]], sem_a))
            handles.append(
                pltpu.async_copy(tok_v.at[j], outt_hbm.at[pos_v.at[j]], sem_b))
        for h in handles:
            h.wait()

    return _sc_scatter


def kernel(x, W, scale, per_expert_scale):
    scale2 = scale.reshape(1, HIDDEN)
    pes2 = per_expert_scale.reshape(1, NEXP)
    keys, vals, hist = pl.pallas_call(
        _router_body,
        grid=(NB1,),
        in_specs=[
            pl.BlockSpec((BT1, HIDDEN), lambda i: (i, 0)),
            pl.BlockSpec((NEXP, HIDDEN), lambda i: (0, 0)),
            pl.BlockSpec((1, HIDDEN), lambda i: (0, 0)),
            pl.BlockSpec((1, NEXP), lambda i: (0, 0)),
        ],
        out_specs=[
            pl.BlockSpec((BT1, TOPK), lambda i: (i, 0)),
            pl.BlockSpec((BT1, TOPK), lambda i: (i, 0)),
            pl.BlockSpec((1, 1, NEXP), lambda i: (i, 0, 0)),
        ],
        out_shape=[
            jax.ShapeDtypeStruct((NTOK, TOPK), jnp.int32),
            jax.ShapeDtypeStruct((NTOK, TOPK), jnp.float32),
            jax.ShapeDtypeStruct((NB1, 1, NEXP), jnp.int32),
        ],
    )(x, W, scale2, pes2)

    pos, cnt = pl.pallas_call(
        _pos_body,
        grid=(NB2,),
        in_specs=[
            pl.BlockSpec((BT2, TOPK), lambda i: (i, 0)),
            pl.BlockSpec((NB1, 1, NEXP), lambda i: (0, 0, 0)),
        ],
        out_specs=[
            pl.BlockSpec((BT2, TOPK), lambda i: (i, 0)),
            pl.BlockSpec((1, NEXP), lambda i: (0, 0)),
        ],
        out_shape=[
            jax.ShapeDtypeStruct((NTOK, TOPK), jnp.int32),
            jax.ShapeDtypeStruct((1, NEXP), jnp.int32),
        ],
        scratch_shapes=[pltpu.VMEM((1, NEXP), jnp.float32)],
        compiler_params=pltpu.CompilerParams(
            dimension_semantics=("arbitrary",)),
    )(keys, hist)

    pos3 = pos.reshape(NW, IDXROWS, 128)
    val3 = vals.reshape(NW, IDXROWS, 128)
    tok3 = (jnp.arange(ITEMS, dtype=jnp.int32) // TOPK).reshape(NW, IDXROWS, 128)
    out_scores, out_tok = _make_sc_scatter()(pos3, val3, tok3)
    return out_scores, out_tok, cnt.reshape(NEXP)


# speedup vs baseline: 1.4458x; 1.1020x over previous
import functools

import jax
import jax.numpy as jnp
from jax import lax
from jax.experimental import pallas as pl
from jax.experimental.pallas import tpu as pltpu
from jax.experimental.pallas import tpu_sc as plsc

HIDDEN = 4096
NEXP = 64
TOPK = 2
NTOK = 16384
EPS = 1e-06

BT1 = 1024
NB1 = NTOK // BT1
BT2 = 512
NB2 = NTOK // BT2
NW = 32
ITEMS = NTOK * TOPK
IPW = ITEMS // NW
IDXROWS = IPW // 128


def _router_body(x_ref, w_ref, scale_ref, pes_ref, keys_ref, vals_ref, hist_ref):
    xb = x_ref[...]
    ms = jnp.mean(xb * xb, axis=1, keepdims=True)
    normed = xb * lax.rsqrt(ms + EPS) * scale_ref[...]
    logits = lax.dot_general(normed, w_ref[...], (((1,), (1,)), ((), ())))
    logits = logits * pes_ref[...]
    scores = jax.nn.sigmoid(logits)
    iot = lax.broadcasted_iota(jnp.int32, scores.shape, 1)
    m1 = jnp.max(scores, axis=1, keepdims=True)
    i1 = jnp.min(jnp.where(scores == m1, iot, NEXP), axis=1, keepdims=True)
    sc2 = jnp.where(iot == i1, -1.0, scores)
    m2 = jnp.max(sc2, axis=1, keepdims=True)
    i2 = jnp.min(jnp.where(sc2 == m2, iot, NEXP), axis=1, keepdims=True)
    keys_ref[...] = jnp.concatenate([i1, i2], axis=1)
    vals_ref[...] = jnp.concatenate([m1, m2], axis=1)
    oh = (iot == i1).astype(jnp.int32) + (iot == i2).astype(jnp.int32)
    hist_ref[...] = jnp.sum(oh, axis=0).reshape(1, 1, NEXP)


def _pos_body(keys_ref, hist_ref, pos_ref, cnt_ref, run_ref):
    b = pl.program_id(0)
    total = jnp.sum(hist_ref[...], axis=(0, 1)).reshape(1, NEXP)
    totf = total.astype(jnp.float32)
    ir = lax.broadcasted_iota(jnp.int32, (NEXP, NEXP), 0)
    ic = lax.broadcasted_iota(jnp.int32, (NEXP, NEXP), 1)
    lmask = (ic < ir).astype(jnp.float32)
    offs = lax.dot_general(totf, lmask, (((1,), (1,)), ((), ())),
                           precision=lax.Precision.HIGHEST)

    @pl.when(b == 0)
    def _():
        run_ref[...] = offs

    run = run_ref[...]
    keys = keys_ref[...]
    k0 = keys[:, 0:1]
    k1 = keys[:, 1:2]
    iot = lax.broadcasted_iota(jnp.int32, (BT2, NEXP), 1)
    oh0 = (iot == k0).astype(jnp.float32)
    oh1 = (iot == k1).astype(jnp.float32)
    ohs = oh0 + oh1
    irb = lax.broadcasted_iota(jnp.int32, (BT2, BT2), 0)
    icb = lax.broadcasted_iota(jnp.int32, (BT2, BT2), 1)
    ltri = (icb < irb).astype(jnp.float32)
    prefix = lax.dot_general(ltri, ohs, (((1,), (0,)), ((), ())),
                             precision=lax.Precision.DEFAULT)
    base = prefix + run
    p0 = jnp.sum(oh0 * base, axis=1, keepdims=True)
    p1 = jnp.sum(oh1 * base, axis=1, keepdims=True)
    pos_ref[...] = jnp.concatenate([p0, p1], axis=1).astype(jnp.int32)
    run_ref[...] = run + jnp.sum(ohs, axis=0, keepdims=True)
    cnt_ref[...] = total


@functools.cache
def _make_sc_scatter():
    mesh = plsc.VectorSubcoreMesh(core_axis_name="c", subcore_axis_name="s")

    @functools.partial(
        pl.kernel,
        mesh=mesh,
        out_type=[
            jax.ShapeDtypeStruct((ITEMS,), jnp.float32),
            jax.ShapeDtypeStruct((ITEMS,), jnp.int32),
        ],
        scratch_types=[
            pltpu.VMEM((IPW,), jnp.int32),
            pltpu.VMEM((IPW,), jnp.float32),
            pltpu.VMEM((IPW,), jnp.int32),
            pltpu.SemaphoreType.DMA,
            pltpu.SemaphoreType.DMA,
        ],
    )
    def _sc_scatter(pos_hbm, val_hbm, tok_hbm, outs_hbm, outt_hbm,
                    pos_v, val_v, tok_v, sem_a, sem_b):
        wid = lax.axis_index("s") * 2 + lax.axis_index("c")
        pltpu.sync_copy(pos_hbm.at[wid], pos_v)
        pltpu.sync_copy(val_hbm.at[wid], val_v)
        pltpu.sync_copy(tok_hbm.at[wid], tok_v)
        ha = pltpu.async_copy(val_v, outs_hbm.at[pos_v], sem_a)
        hb = pltpu.async_copy(tok_v, outt_hbm.at[pos_v], sem_b)
        ha.wait()
        hb.wait()

    return _sc_scatter


def kernel(x, W, scale, per_expert_scale):
    scale2 = scale.reshape(1, HIDDEN)
    pes2 = per_expert_scale.reshape(1, NEXP)
    keys, vals, hist = pl.pallas_call(
        _router_body,
        grid=(NB1,),
        in_specs=[
            pl.BlockSpec((BT1, HIDDEN), lambda i: (i, 0)),
            pl.BlockSpec((NEXP, HIDDEN), lambda i: (0, 0)),
            pl.BlockSpec((1, HIDDEN), lambda i: (0, 0)),
            pl.BlockSpec((1, NEXP), lambda i: (0, 0)),
        ],
        out_specs=[
            pl.BlockSpec((BT1, TOPK), lambda i: (i, 0)),
            pl.BlockSpec((BT1, TOPK), lambda i: (i, 0)),
            pl.BlockSpec((1, 1, NEXP), lambda i: (i, 0, 0)),
        ],
        out_shape=[
            jax.ShapeDtypeStruct((NTOK, TOPK), jnp.int32),
            jax.ShapeDtypeStruct((NTOK, TOPK), jnp.float32),
            jax.ShapeDtypeStruct((NB1, 1, NEXP), jnp.int32),
        ],
    )(x, W, scale2, pes2)

    pos, cnt = pl.pallas_call(
        _pos_body,
        grid=(NB2,),
        in_specs=[
            pl.BlockSpec((BT2, TOPK), lambda i: (i, 0)),
            pl.BlockSpec((NB1, 1, NEXP), lambda i: (0, 0, 0)),
        ],
        out_specs=[
            pl.BlockSpec((BT2, TOPK), lambda i: (i, 0)),
            pl.BlockSpec((1, NEXP), lambda i: (0, 0)),
        ],
        out_shape=[
            jax.ShapeDtypeStruct((NTOK, TOPK), jnp.int32),
            jax.ShapeDtypeStruct((1, NEXP), jnp.int32),
        ],
        scratch_shapes=[pltpu.VMEM((1, NEXP), jnp.float32)],
        compiler_params=pltpu.CompilerParams(
            dimension_semantics=("arbitrary",)),
    )(keys, hist)

    pos3 = pos.reshape(NW, IPW)
    val3 = vals.reshape(NW, IPW)
    tok3 = (jnp.arange(ITEMS, dtype=jnp.int32) // TOPK).reshape(NW, IPW)
    out_scores, out_tok = _make_sc_scatter()(pos3, val3, tok3)
    return out_scores, out_tok, cnt.reshape(NEXP)


# speedup vs baseline: 2.1386x; 1.4792x over previous
import functools

import jax
import jax.numpy as jnp
from jax import lax
from jax.experimental import pallas as pl
from jax.experimental.pallas import tpu as pltpu
from jax.experimental.pallas import tpu_sc as plsc

HIDDEN = 4096
NEXP = 64
TOPK = 2
NTOK = 16384
EPS = 1e-06

BT1 = 1024
NB1 = NTOK // BT1
BT2 = 512
NB2 = NTOK // BT2
NW = 32
ITEMS = NTOK * TOPK
IPC = ITEMS // 16


def _router_body(x_ref, w_ref, scale_ref, pes_ref, keys_ref, vals_ref, hist_ref):
    xb = x_ref[...]
    ms = jnp.mean(xb * xb, axis=1, keepdims=True)
    normed = xb * lax.rsqrt(ms + EPS) * scale_ref[...]
    logits = lax.dot_general(normed, w_ref[...], (((1,), (1,)), ((), ())))
    logits = logits * pes_ref[...]
    scores = jax.nn.sigmoid(logits)
    iot = lax.broadcasted_iota(jnp.int32, scores.shape, 1)
    m1 = jnp.max(scores, axis=1, keepdims=True)
    i1 = jnp.min(jnp.where(scores == m1, iot, NEXP), axis=1, keepdims=True)
    sc2 = jnp.where(iot == i1, -1.0, scores)
    m2 = jnp.max(sc2, axis=1, keepdims=True)
    i2 = jnp.min(jnp.where(sc2 == m2, iot, NEXP), axis=1, keepdims=True)
    keys_ref[...] = jnp.concatenate([i1, i2], axis=1)
    vals_ref[...] = jnp.concatenate([m1, m2], axis=1)
    oh = (iot == i1).astype(jnp.int32) + (iot == i2).astype(jnp.int32)
    hist_ref[...] = jnp.sum(oh, axis=0).reshape(1, 1, NEXP)


def _pos_body(keys_ref, hist_ref, pos_ref, cnt_ref, run_ref):
    b = pl.program_id(0)
    total = jnp.sum(hist_ref[...], axis=(0, 1)).reshape(1, NEXP)
    totf = total.astype(jnp.float32)
    ir = lax.broadcasted_iota(jnp.int32, (NEXP, NEXP), 0)
    ic = lax.broadcasted_iota(jnp.int32, (NEXP, NEXP), 1)
    lmask = (ic < ir).astype(jnp.float32)
    offs = lax.dot_general(totf, lmask, (((1,), (1,)), ((), ())),
                           precision=lax.Precision.HIGHEST)

    @pl.when(b == 0)
    def _():
        run_ref[...] = offs

    run = run_ref[...]
    keys = keys_ref[...]
    k0 = keys[:, 0:1]
    k1 = keys[:, 1:2]
    iot = lax.broadcasted_iota(jnp.int32, (BT2, NEXP), 1)
    oh0 = (iot == k0).astype(jnp.float32)
    oh1 = (iot == k1).astype(jnp.float32)
    ohs = oh0 + oh1
    irb = lax.broadcasted_iota(jnp.int32, (BT2, BT2), 0)
    icb = lax.broadcasted_iota(jnp.int32, (BT2, BT2), 1)
    ltri = (icb < irb).astype(jnp.float32)
    prefix = lax.dot_general(ltri, ohs, (((1,), (0,)), ((), ())),
                             precision=lax.Precision.DEFAULT)
    base = prefix + run
    p0 = jnp.sum(oh0 * base, axis=1, keepdims=True)
    p1 = jnp.sum(oh1 * base, axis=1, keepdims=True)
    pos_ref[...] = jnp.concatenate([p0, p1], axis=1).astype(jnp.int32)
    run_ref[...] = run + jnp.sum(ohs, axis=0, keepdims=True)
    cnt_ref[...] = total


@functools.cache
def _make_sc_scatter():
    mesh = plsc.VectorSubcoreMesh(core_axis_name="c", subcore_axis_name="s")

    @functools.partial(
        pl.kernel,
        mesh=mesh,
        out_type=[
            jax.ShapeDtypeStruct((ITEMS,), jnp.int32),
            jax.ShapeDtypeStruct((ITEMS,), jnp.int32),
        ],
        scratch_types=[
            pltpu.VMEM((IPC,), jnp.int32),
            pltpu.VMEM((IPC,), jnp.int32),
            pltpu.VMEM_SHARED((ITEMS,), jnp.int32),
        ],
    )
    def _sc_scatter(pos_hbm, val_hbm, tok_hbm, outs_hbm, outt_hbm,
                    pos_v, dat_v, sh):
        cid = lax.axis_index("c")
        sid = lax.axis_index("s")
        pltpu.sync_copy(pos_hbm.at[sid], pos_v)

        @pl.when(cid == 0)
        def _():
            pltpu.sync_copy(val_hbm.at[sid], dat_v)

        @pl.when(cid == 1)
        def _():
            pltpu.sync_copy(tok_hbm.at[sid], dat_v)

        pltpu.sync_copy(dat_v, sh.at[pos_v])
        plsc.subcore_barrier()

        @pl.when((cid == 0) & (sid == 0))
        def _():
            pltpu.sync_copy(sh, outs_hbm)

        @pl.when((cid == 1) & (sid == 0))
        def _():
            pltpu.sync_copy(sh, outt_hbm)

    return _sc_scatter


def kernel(x, W, scale, per_expert_scale):
    scale2 = scale.reshape(1, HIDDEN)
    pes2 = per_expert_scale.reshape(1, NEXP)
    keys, vals, hist = pl.pallas_call(
        _router_body,
        grid=(NB1,),
        in_specs=[
            pl.BlockSpec((BT1, HIDDEN), lambda i: (i, 0)),
            pl.BlockSpec((NEXP, HIDDEN), lambda i: (0, 0)),
            pl.BlockSpec((1, HIDDEN), lambda i: (0, 0)),
            pl.BlockSpec((1, NEXP), lambda i: (0, 0)),
        ],
        out_specs=[
            pl.BlockSpec((BT1, TOPK), lambda i: (i, 0)),
            pl.BlockSpec((BT1, TOPK), lambda i: (i, 0)),
            pl.BlockSpec((1, 1, NEXP), lambda i: (i, 0, 0)),
        ],
        out_shape=[
            jax.ShapeDtypeStruct((NTOK, TOPK), jnp.int32),
            jax.ShapeDtypeStruct((NTOK, TOPK), jnp.float32),
            jax.ShapeDtypeStruct((NB1, 1, NEXP), jnp.int32),
        ],
    )(x, W, scale2, pes2)

    pos, cnt = pl.pallas_call(
        _pos_body,
        grid=(NB2,),
        in_specs=[
            pl.BlockSpec((BT2, TOPK), lambda i: (i, 0)),
            pl.BlockSpec((NB1, 1, NEXP), lambda i: (0, 0, 0)),
        ],
        out_specs=[
            pl.BlockSpec((BT2, TOPK), lambda i: (i, 0)),
            pl.BlockSpec((1, NEXP), lambda i: (0, 0)),
        ],
        out_shape=[
            jax.ShapeDtypeStruct((NTOK, TOPK), jnp.int32),
            jax.ShapeDtypeStruct((1, NEXP), jnp.int32),
        ],
        scratch_shapes=[pltpu.VMEM((1, NEXP), jnp.float32)],
        compiler_params=pltpu.CompilerParams(
            dimension_semantics=("arbitrary",)),
    )(keys, hist)

    pos3 = pos.reshape(16, IPC)
    val3 = lax.bitcast_convert_type(vals, jnp.int32).reshape(16, IPC)
    tok3 = (jnp.arange(ITEMS, dtype=jnp.int32) // TOPK).reshape(16, IPC)
    outs_i, out_tok = _make_sc_scatter()(pos3, val3, tok3)
    out_scores = lax.bitcast_convert_type(outs_i, jnp.float32)
    return out_scores, out_tok, cnt.reshape(NEXP)


# speedup vs baseline: 2.6951x; 1.2602x over previous
import functools

import jax
import jax.numpy as jnp
from jax import lax
from jax.experimental import pallas as pl
from jax.experimental.pallas import tpu as pltpu
from jax.experimental.pallas import tpu_sc as plsc

HIDDEN = 4096
NEXP = 64
TOPK = 2
NTOK = 16384
EPS = 1e-06

BT1 = 1024
NB1 = NTOK // BT1
ITEMS = NTOK * TOPK
IPC = ITEMS // 16


def _router_body(x_ref, w_ref, scale_ref, pes_ref,
                 rk_ref, val_ref, offs_ref, cnt_ref, run_ref):
    b = pl.program_id(0)

    @pl.when(b == 0)
    def _():
        run_ref[...] = jnp.zeros_like(run_ref)

    xb = x_ref[...]
    ms = jnp.mean(xb * xb, axis=1, keepdims=True)
    normed = xb * lax.rsqrt(ms + EPS) * scale_ref[...]
    logits = lax.dot_general(normed, w_ref[...], (((1,), (1,)), ((), ())))
    logits = logits * pes_ref[...]
    scores = jax.nn.sigmoid(logits)
    iot = lax.broadcasted_iota(jnp.int32, scores.shape, 1)
    m1 = jnp.max(scores, axis=1, keepdims=True)
    i1 = jnp.min(jnp.where(scores == m1, iot, NEXP), axis=1, keepdims=True)
    sc2 = jnp.where(iot == i1, -1.0, scores)
    m2 = jnp.max(sc2, axis=1, keepdims=True)
    i2 = jnp.min(jnp.where(sc2 == m2, iot, NEXP), axis=1, keepdims=True)

    oh0 = (iot == i1).astype(jnp.float32)
    oh1 = (iot == i2).astype(jnp.float32)
    ohs = oh0 + oh1
    irb = lax.broadcasted_iota(jnp.int32, (BT1, BT1), 0)
    icb = lax.broadcasted_iota(jnp.int32, (BT1, BT1), 1)
    ltri = (icb < irb).astype(jnp.bfloat16)
    prefix = lax.dot_general(ltri, ohs.astype(jnp.bfloat16),
                             (((1,), (0,)), ((), ())),
                             preferred_element_type=jnp.float32)
    run = run_ref[...]
    base = prefix + run
    r0 = jnp.sum(oh0 * base, axis=1, keepdims=True)
    r1 = jnp.sum(oh1 * base, axis=1, keepdims=True)
    newrun = run + jnp.sum(ohs, axis=0, keepdims=True)
    run_ref[...] = newrun

    rk0 = r0.astype(jnp.int32) * NEXP + i1
    rk1 = r1.astype(jnp.int32) * NEXP + i2
    rk_ref[...] = jnp.transpose(jnp.concatenate([rk0, rk1], axis=1), (1, 0)).reshape(1, 2, BT1)
    v01 = jnp.concatenate([m1, m2], axis=1)
    val_ref[...] = jnp.transpose(lax.bitcast_convert_type(v01, jnp.int32), (1, 0)).reshape(1, 2, BT1)

    ir = lax.broadcasted_iota(jnp.int32, (NEXP, NEXP), 0)
    ic = lax.broadcasted_iota(jnp.int32, (NEXP, NEXP), 1)
    lmask = (ic < ir).astype(jnp.float32)
    offs = lax.dot_general(newrun, lmask, (((1,), (1,)), ((), ())),
                           precision=lax.Precision.HIGHEST)
    offs_ref[...] = offs.astype(jnp.int32)
    cnt_ref[...] = newrun.astype(jnp.int32)


@functools.cache
def _make_sc_scatter():
    mesh = plsc.VectorSubcoreMesh(core_axis_name="c", subcore_axis_name="s")

    @functools.partial(
        pl.kernel,
        mesh=mesh,
        out_type=[
            jax.ShapeDtypeStruct((ITEMS,), jnp.int32),
            jax.ShapeDtypeStruct((ITEMS,), jnp.int32),
        ],
        scratch_types=[
            pltpu.VMEM((1, NEXP), jnp.int32),
            pltpu.VMEM((NEXP,), jnp.int32),
            pltpu.VMEM((2, BT1), jnp.int32),
            pltpu.VMEM((2, BT1), jnp.int32),
            pltpu.VMEM((IPC,), jnp.int32),
            pltpu.VMEM((IPC,), jnp.int32),
            pltpu.VMEM_SHARED((ITEMS,), jnp.int32),
        ],
        compiler_params=pltpu.CompilerParams(needs_layout_passes=False),
    )
    def _sc_scatter(rk_hbm, val_hbm, offs_hbm, outs_hbm, outt_hbm,
                    offs2_v, offs_v, rk_v, valr_v, pos_v, dat_v, sh):
        cid = lax.axis_index("c")
        sid = lax.axis_index("s")
        pltpu.sync_copy(offs_hbm, offs2_v)
        pltpu.sync_copy(rk_hbm.at[sid], rk_v)

        @pl.when(cid == 0)
        def _():
            pltpu.sync_copy(val_hbm.at[sid], valr_v)

        for c in range(4):
            offs_v[pl.ds(c * 16, 16)] = offs2_v[0, pl.ds(c * 16, 16)]

        tbase = sid * BT1
        lane = lax.broadcasted_iota(jnp.int32, (16,), 0)
        for c in range(IPC // 16):
            col = (c % 64) * 16
            rk = rk_v[c // 64, pl.ds(col, 16)]
            key = lax.rem(rk, NEXP)
            rank = lax.div(rk, NEXP)
            off = plsc.load_gather(offs_v, [key])
            pos_v[pl.ds(c * 16, 16)] = off + rank

            @pl.when(cid == 0)
            def _():
                dat_v[pl.ds(c * 16, 16)] = valr_v[c // 64, pl.ds(col, 16)]

            @pl.when(cid == 1)
            def _():
                dat_v[pl.ds(c * 16, 16)] = tbase + col + lane

        pltpu.sync_copy(dat_v, sh.at[pos_v])
        plsc.subcore_barrier()

        @pl.when((cid == 0) & (sid == 0))
        def _():
            pltpu.sync_copy(sh, outs_hbm)

        @pl.when((cid == 1) & (sid == 0))
        def _():
            pltpu.sync_copy(sh, outt_hbm)

    return _sc_scatter


def kernel(x, W, scale, per_expert_scale):
    scale2 = scale.reshape(1, HIDDEN)
    pes2 = per_expert_scale.reshape(1, NEXP)
    rk, val, offs, cnt = pl.pallas_call(
        _router_body,
        grid=(NB1,),
        in_specs=[
            pl.BlockSpec((BT1, HIDDEN), lambda i: (i, 0)),
            pl.BlockSpec((NEXP, HIDDEN), lambda i: (0, 0)),
            pl.BlockSpec((1, HIDDEN), lambda i: (0, 0)),
            pl.BlockSpec((1, NEXP), lambda i: (0, 0)),
        ],
        out_specs=[
            pl.BlockSpec((1, 2, BT1), lambda i: (i, 0, 0)),
            pl.BlockSpec((1, 2, BT1), lambda i: (i, 0, 0)),
            pl.BlockSpec((1, NEXP), lambda i: (0, 0)),
            pl.BlockSpec((1, NEXP), lambda i: (0, 0)),
        ],
        out_shape=[
            jax.ShapeDtypeStruct((NB1, 2, BT1), jnp.int32),
            jax.ShapeDtypeStruct((NB1, 2, BT1), jnp.int32),
            jax.ShapeDtypeStruct((1, NEXP), jnp.int32),
            jax.ShapeDtypeStruct((1, NEXP), jnp.int32),
        ],
        scratch_shapes=[pltpu.VMEM((1, NEXP), jnp.float32)],
        compiler_params=pltpu.CompilerParams(
            dimension_semantics=("arbitrary",)),
    )(x, W, scale2, pes2)

    outs_i, out_tok = _make_sc_scatter()(rk, val, offs)
    out_scores = lax.bitcast_convert_type(outs_i, jnp.float32)
    return out_scores, out_tok, cnt.reshape(NEXP)


# speedup vs baseline: 3.1295x; 1.1612x over previous
import functools

import jax
import jax.numpy as jnp
from jax import lax
from jax.experimental import pallas as pl
from jax.experimental.pallas import tpu as pltpu
from jax.experimental.pallas import tpu_sc as plsc

HIDDEN = 4096
NEXP = 64
TOPK = 2
NTOK = 16384
EPS = 1e-06

BT1 = 1024
NB1 = NTOK // BT1
ITEMS = NTOK * TOPK
IPC = ITEMS // 16


def _router_body(x_ref, w_ref, scale_ref, pes_ref, utri_ref,
                 rk_ref, val_ref, offs_ref, cnt_ref, run_ref):
    b = pl.program_id(0)

    @pl.when(b == 0)
    def _():
        run_ref[...] = jnp.zeros_like(run_ref)

    xb = x_ref[...]
    ms = jnp.mean(xb * xb, axis=1, keepdims=True)
    normed = xb * lax.rsqrt(ms + EPS) * scale_ref[...]
    logits = lax.dot_general(w_ref[...], normed, (((1,), (1,)), ((), ())))
    logits = logits * pes_ref[...]
    scores = jax.nn.sigmoid(logits)
    iot = lax.broadcasted_iota(jnp.int32, scores.shape, 0)
    m1 = jnp.max(scores, axis=0, keepdims=True)
    i1 = jnp.min(jnp.where(scores == m1, iot, NEXP), axis=0, keepdims=True)
    sc2 = jnp.where(iot == i1, -1.0, scores)
    m2 = jnp.max(sc2, axis=0, keepdims=True)
    i2 = jnp.min(jnp.where(sc2 == m2, iot, NEXP), axis=0, keepdims=True)

    oh0 = (iot == i1).astype(jnp.float32)
    oh1 = (iot == i2).astype(jnp.float32)
    ohs = oh0 + oh1
    prefix = lax.dot_general(ohs.astype(jnp.bfloat16), utri_ref[...],
                             (((1,), (0,)), ((), ())),
                             preferred_element_type=jnp.float32)
    run = run_ref[...]
    base = prefix + run
    r0 = jnp.sum(oh0 * base, axis=0, keepdims=True)
    r1 = jnp.sum(oh1 * base, axis=0, keepdims=True)
    newrun = run + jnp.sum(ohs, axis=1, keepdims=True)
    run_ref[...] = newrun

    rk0 = r0.astype(jnp.int32) * NEXP + i1
    rk1 = r1.astype(jnp.int32) * NEXP + i2
    rk_ref[...] = jnp.concatenate([rk0, rk1], axis=0).reshape(1, 2, BT1)
    v01 = jnp.concatenate([m1, m2], axis=0)
    val_ref[...] = lax.bitcast_convert_type(v01, jnp.int32).reshape(1, 2, BT1)

    ir = lax.broadcasted_iota(jnp.int32, (NEXP, NEXP), 0)
    ic = lax.broadcasted_iota(jnp.int32, (NEXP, NEXP), 1)
    umask = (ir < ic).astype(jnp.float32)
    eye = (ir == ic).astype(jnp.float32)
    offs = lax.dot_general(newrun, umask, (((0,), (0,)), ((), ())),
                           precision=lax.Precision.HIGHEST)
    offs_ref[...] = offs.astype(jnp.int32)
    cnt = lax.dot_general(newrun, eye, (((0,), (0,)), ((), ())),
                          precision=lax.Precision.HIGHEST)
    cnt_ref[...] = cnt.astype(jnp.int32)


@functools.cache
def _make_sc_scatter():
    mesh = plsc.VectorSubcoreMesh(core_axis_name="c", subcore_axis_name="s")

    @functools.partial(
        pl.kernel,
        mesh=mesh,
        out_type=[
            jax.ShapeDtypeStruct((ITEMS,), jnp.int32),
            jax.ShapeDtypeStruct((ITEMS,), jnp.int32),
        ],
        scratch_types=[
            pltpu.VMEM((1, NEXP), jnp.int32),
            pltpu.VMEM((NEXP,), jnp.int32),
            pltpu.VMEM((2, BT1), jnp.int32),
            pltpu.VMEM((2, BT1), jnp.int32),
            pltpu.VMEM((IPC,), jnp.int32),
            pltpu.VMEM((IPC,), jnp.int32),
            pltpu.VMEM_SHARED((ITEMS,), jnp.int32),
        ],
        compiler_params=pltpu.CompilerParams(needs_layout_passes=False),
    )
    def _sc_scatter(rk_hbm, val_hbm, offs_hbm, outs_hbm, outt_hbm,
                    offs2_v, offs_v, rk_v, valr_v, pos_v, dat_v, sh):
        cid = lax.axis_index("c")
        sid = lax.axis_index("s")
        pltpu.sync_copy(offs_hbm, offs2_v)
        pltpu.sync_copy(rk_hbm.at[sid], rk_v)

        @pl.when(cid == 0)
        def _():
            pltpu.sync_copy(val_hbm.at[sid], valr_v)

        for c in range(4):
            offs_v[pl.ds(c * 16, 16)] = offs2_v[0, pl.ds(c * 16, 16)]

        tbase = sid * BT1
        lane = lax.broadcasted_iota(jnp.int32, (16,), 0)
        for c in range(IPC // 16):
            col = (c % 64) * 16
            rk = rk_v[c // 64, pl.ds(col, 16)]
            key = lax.rem(rk, NEXP)
            rank = lax.div(rk, NEXP)
            off = plsc.load_gather(offs_v, [key])
            pos_v[pl.ds(c * 16, 16)] = off + rank

            @pl.when(cid == 0)
            def _():
                dat_v[pl.ds(c * 16, 16)] = valr_v[c // 64, pl.ds(col, 16)]

            @pl.when(cid == 1)
            def _():
                dat_v[pl.ds(c * 16, 16)] = tbase + col + lane

        pltpu.sync_copy(dat_v, sh.at[pos_v])
        plsc.subcore_barrier()

        @pl.when((cid == 0) & (sid == 0))
        def _():
            pltpu.sync_copy(sh, outs_hbm)

        @pl.when((cid == 1) & (sid == 0))
        def _():
            pltpu.sync_copy(sh, outt_hbm)

    return _sc_scatter


def kernel(x, W, scale, per_expert_scale):
    scale2 = scale.reshape(1, HIDDEN)
    pes2 = per_expert_scale.reshape(NEXP, 1)
    utri = jnp.triu(jnp.ones((BT1, BT1), jnp.bfloat16), 1)
    rk, val, offs, cnt = pl.pallas_call(
        _router_body,
        grid=(NB1,),
        in_specs=[
            pl.BlockSpec((BT1, HIDDEN), lambda i: (i, 0)),
            pl.BlockSpec((NEXP, HIDDEN), lambda i: (0, 0)),
            pl.BlockSpec((1, HIDDEN), lambda i: (0, 0)),
            pl.BlockSpec((NEXP, 1), lambda i: (0, 0)),
            pl.BlockSpec((BT1, BT1), lambda i: (0, 0)),
        ],
        out_specs=[
            pl.BlockSpec((1, 2, BT1), lambda i: (i, 0, 0)),
            pl.BlockSpec((1, 2, BT1), lambda i: (i, 0, 0)),
            pl.BlockSpec((1, NEXP), lambda i: (0, 0)),
            pl.BlockSpec((1, NEXP), lambda i: (0, 0)),
        ],
        out_shape=[
            jax.ShapeDtypeStruct((NB1, 2, BT1), jnp.int32),
            jax.ShapeDtypeStruct((NB1, 2, BT1), jnp.int32),
            jax.ShapeDtypeStruct((1, NEXP), jnp.int32),
            jax.ShapeDtypeStruct((1, NEXP), jnp.int32),
        ],
        scratch_shapes=[pltpu.VMEM((NEXP, 1), jnp.float32)],
        compiler_params=pltpu.CompilerParams(
            dimension_semantics=("arbitrary",)),
    )(x, W, scale2, pes2, utri)

    outs_i, out_tok = _make_sc_scatter()(rk, val, offs)
    out_scores = lax.bitcast_convert_type(outs_i, jnp.float32)
    return out_scores, out_tok, cnt.reshape(NEXP)


# speedup vs baseline: 3.1855x; 1.0179x over previous
import functools

import jax
import jax.numpy as jnp
import numpy as np
from jax import lax
from jax.experimental import pallas as pl
from jax.experimental.pallas import tpu as pltpu
from jax.experimental.pallas import tpu_sc as plsc

HIDDEN = 4096
NEXP = 64
TOPK = 2
NTOK = 16384
EPS = 1e-06

BT1 = 1024
NB1 = NTOK // BT1
ITEMS = NTOK * TOPK
IPC = ITEMS // 16


_UTRI = np.triu(np.ones((BT1, BT1), np.float32), 1).astype(jnp.bfloat16)


def _router_body(x_ref, w_ref, scale_ref, pes_ref, utri_ref,
                 rk_ref, val_ref, offs_ref, cnt_ref, run_ref):
    b = pl.program_id(0)

    @pl.when(b == 0)
    def _():
        run_ref[...] = jnp.zeros_like(run_ref)

    xb = x_ref[...]
    ms = jnp.mean(xb * xb, axis=1, keepdims=True)
    normed = xb * lax.rsqrt(ms + EPS) * scale_ref[...]
    logits = lax.dot_general(w_ref[...], normed, (((1,), (1,)), ((), ())))
    logits = logits * pes_ref[...]
    scores = jax.nn.sigmoid(logits)
    iot = lax.broadcasted_iota(jnp.int32, scores.shape, 0)
    m1 = jnp.max(scores, axis=0, keepdims=True)
    i1 = jnp.min(jnp.where(scores == m1, iot, NEXP), axis=0, keepdims=True)
    sc2 = jnp.where(iot == i1, -1.0, scores)
    m2 = jnp.max(sc2, axis=0, keepdims=True)
    i2 = jnp.min(jnp.where(sc2 == m2, iot, NEXP), axis=0, keepdims=True)

    oh0 = (iot == i1).astype(jnp.float32)
    oh1 = (iot == i2).astype(jnp.float32)
    ohs = oh0 + oh1
    prefix = lax.dot_general(ohs.astype(jnp.bfloat16), utri_ref[...],
                             (((1,), (0,)), ((), ())),
                             preferred_element_type=jnp.float32)
    run = run_ref[...]
    base = prefix + run
    r0 = jnp.sum(oh0 * base, axis=0, keepdims=True)
    r1 = jnp.sum(oh1 * base, axis=0, keepdims=True)
    newrun = run + jnp.sum(ohs, axis=1, keepdims=True)
    run_ref[...] = newrun

    rk0 = r0.astype(jnp.int32) * NEXP + i1
    rk1 = r1.astype(jnp.int32) * NEXP + i2
    rk_ref[...] = jnp.concatenate([rk0, rk1], axis=0).reshape(1, 2, BT1)
    val_ref[...] = jnp.concatenate([m1, m2], axis=0).reshape(1, 2, BT1)

    ir = lax.broadcasted_iota(jnp.int32, (NEXP, NEXP), 0)
    ic = lax.broadcasted_iota(jnp.int32, (NEXP, NEXP), 1)
    umask = (ir < ic).astype(jnp.float32)
    eye = (ir == ic).astype(jnp.float32)
    offs = lax.dot_general(newrun, umask, (((0,), (0,)), ((), ())),
                           precision=lax.Precision.HIGHEST)
    offs_ref[...] = offs.astype(jnp.int32)
    cnt = lax.dot_general(newrun, eye, (((0,), (0,)), ((), ())),
                          precision=lax.Precision.HIGHEST)
    cnt_ref[...] = cnt.astype(jnp.int32)


@functools.cache
def _make_sc_scatter():
    mesh = plsc.VectorSubcoreMesh(core_axis_name="c", subcore_axis_name="s")

    @functools.partial(
        pl.kernel,
        mesh=mesh,
        out_type=[
            jax.ShapeDtypeStruct((ITEMS,), jnp.float32),
            jax.ShapeDtypeStruct((ITEMS,), jnp.int32),
        ],
        scratch_types=[
            pltpu.VMEM((1, NEXP), jnp.int32),
            pltpu.VMEM((NEXP,), jnp.int32),
            pltpu.VMEM((2, BT1), jnp.int32),
            pltpu.VMEM((2, BT1), jnp.float32),
            pltpu.VMEM((IPC,), jnp.int32),
            pltpu.VMEM((IPC,), jnp.float32),
            pltpu.VMEM((IPC,), jnp.int32),
            pltpu.VMEM_SHARED((ITEMS,), jnp.float32),
            pltpu.VMEM_SHARED((ITEMS,), jnp.int32),
        ],
        compiler_params=pltpu.CompilerParams(needs_layout_passes=False),
    )
    def _sc_scatter(rk_hbm, val_hbm, offs_hbm, outs_hbm, outt_hbm,
                    offs2_v, offs_v, rk_v, valr_v, pos_v, datf_v, dati_v,
                    sh_f, sh_i):
        cid = lax.axis_index("c")
        sid = lax.axis_index("s")
        pltpu.sync_copy(offs_hbm, offs2_v)
        pltpu.sync_copy(rk_hbm.at[sid], rk_v)

        @pl.when(cid == 0)
        def _():
            pltpu.sync_copy(val_hbm.at[sid], valr_v)

        for c in range(4):
            offs_v[pl.ds(c * 16, 16)] = offs2_v[0, pl.ds(c * 16, 16)]

        tbase = sid * BT1
        lane = lax.broadcasted_iota(jnp.int32, (16,), 0)
        for c in range(IPC // 16):
            col = (c % 64) * 16
            rk = rk_v[c // 64, pl.ds(col, 16)]
            key = lax.rem(rk, NEXP)
            rank = lax.div(rk, NEXP)
            off = plsc.load_gather(offs_v, [key])
            pos_v[pl.ds(c * 16, 16)] = off + rank

            @pl.when(cid == 0)
            def _():
                datf_v[pl.ds(c * 16, 16)] = valr_v[c // 64, pl.ds(col, 16)]

            @pl.when(cid == 1)
            def _():
                dati_v[pl.ds(c * 16, 16)] = tbase + col + lane

        @pl.when(cid == 0)
        def _():
            pltpu.sync_copy(datf_v, sh_f.at[pos_v])

        @pl.when(cid == 1)
        def _():
            pltpu.sync_copy(dati_v, sh_i.at[pos_v])

        plsc.subcore_barrier()

        @pl.when((cid == 0) & (sid == 0))
        def _():
            pltpu.sync_copy(sh_f, outs_hbm)

        @pl.when((cid == 1) & (sid == 0))
        def _():
            pltpu.sync_copy(sh_i, outt_hbm)

    return _sc_scatter


def kernel(x, W, scale, per_expert_scale):
    scale2 = scale.reshape(1, HIDDEN)
    pes2 = per_expert_scale.reshape(NEXP, 1)
    utri = _UTRI
    rk, val, offs, cnt = pl.pallas_call(
        _router_body,
        grid=(NB1,),
        in_specs=[
            pl.BlockSpec((BT1, HIDDEN), lambda i: (i, 0)),
            pl.BlockSpec((NEXP, HIDDEN), lambda i: (0, 0)),
            pl.BlockSpec((1, HIDDEN), lambda i: (0, 0)),
            pl.BlockSpec((NEXP, 1), lambda i: (0, 0)),
            pl.BlockSpec((BT1, BT1), lambda i: (0, 0)),
        ],
        out_specs=[
            pl.BlockSpec((1, 2, BT1), lambda i: (i, 0, 0)),
            pl.BlockSpec((1, 2, BT1), lambda i: (i, 0, 0)),
            pl.BlockSpec((1, NEXP), lambda i: (0, 0)),
            pl.BlockSpec((1, NEXP), lambda i: (0, 0)),
        ],
        out_shape=[
            jax.ShapeDtypeStruct((NB1, 2, BT1), jnp.int32),
            jax.ShapeDtypeStruct((NB1, 2, BT1), jnp.float32),
            jax.ShapeDtypeStruct((1, NEXP), jnp.int32),
            jax.ShapeDtypeStruct((1, NEXP), jnp.int32),
        ],
        scratch_shapes=[pltpu.VMEM((NEXP, 1), jnp.float32)],
        compiler_params=pltpu.CompilerParams(
            dimension_semantics=("arbitrary",)),
    )(x, W, scale2, pes2, utri)

    out_scores, out_tok = _make_sc_scatter()(rk, val, offs)
    return out_scores, out_tok, cnt.reshape(NEXP)


# speedup vs baseline: 3.2341x; 1.0152x over previous
import functools

import jax
import jax.numpy as jnp
import numpy as np
from jax import lax
from jax.experimental import pallas as pl
from jax.experimental.pallas import tpu as pltpu
from jax.experimental.pallas import tpu_sc as plsc

HIDDEN = 4096
NEXP = 64
TOPK = 2
NTOK = 16384
EPS = 1e-06

BT1 = 1024
NB1 = NTOK // BT1
ITEMS = NTOK * TOPK
IPC = ITEMS // 16


CH = 256
_UTRI = np.triu(np.ones((CH, CH), np.float32), 1).astype(jnp.bfloat16)


def _router_body(x_ref, w_ref, scale_ref, pes_ref, utri_ref,
                 rk_ref, val_ref, offs_ref, cnt_ref, run_ref):
    b = pl.program_id(0)

    @pl.when(b == 0)
    def _():
        run_ref[...] = jnp.zeros_like(run_ref)

    xb = x_ref[...]
    ms = jnp.mean(xb * xb, axis=1, keepdims=True)
    normed = xb * lax.rsqrt(ms + EPS) * scale_ref[...]
    logits = lax.dot_general(w_ref[...], normed, (((1,), (1,)), ((), ())))
    logits = logits * pes_ref[...]
    scores = jax.nn.sigmoid(logits)
    iot = lax.broadcasted_iota(jnp.int32, scores.shape, 0)
    m1 = jnp.max(scores, axis=0, keepdims=True)
    i1 = jnp.min(jnp.where(scores == m1, iot, NEXP), axis=0, keepdims=True)
    sc2 = jnp.where(iot == i1, -1.0, scores)
    m2 = jnp.max(sc2, axis=0, keepdims=True)
    i2 = jnp.min(jnp.where(sc2 == m2, iot, NEXP), axis=0, keepdims=True)

    oh0 = (iot == i1).astype(jnp.float32)
    oh1 = (iot == i2).astype(jnp.float32)
    ohs = oh0 + oh1
    ohs_bf = ohs.astype(jnp.bfloat16)
    utri = utri_ref[...]
    parts = []
    carry = jnp.zeros((NEXP, 1), jnp.float32)
    for k in range(BT1 // CH):
        seg = ohs_bf[:, k * CH:(k + 1) * CH]
        p = lax.dot_general(seg, utri, (((1,), (0,)), ((), ())),
                            preferred_element_type=jnp.float32)
        parts.append(p + carry)
        carry = carry + jnp.sum(ohs[:, k * CH:(k + 1) * CH],
                                axis=1, keepdims=True)
    prefix = jnp.concatenate(parts, axis=1)
    run = run_ref[...]
    base = prefix + run
    r0 = jnp.sum(oh0 * base, axis=0, keepdims=True)
    r1 = jnp.sum(oh1 * base, axis=0, keepdims=True)
    newrun = run + jnp.sum(ohs, axis=1, keepdims=True)
    run_ref[...] = newrun

    rk0 = r0.astype(jnp.int32) * NEXP + i1
    rk1 = r1.astype(jnp.int32) * NEXP + i2
    rk_ref[...] = jnp.concatenate([rk0, rk1], axis=0).reshape(1, 2, BT1)
    val_ref[...] = jnp.concatenate([m1, m2], axis=0).reshape(1, 2, BT1)

    ir = lax.broadcasted_iota(jnp.int32, (NEXP, NEXP), 0)
    ic = lax.broadcasted_iota(jnp.int32, (NEXP, NEXP), 1)
    umask = (ir < ic).astype(jnp.float32)
    eye = (ir == ic).astype(jnp.float32)
    offs = lax.dot_general(newrun, umask, (((0,), (0,)), ((), ())),
                           precision=lax.Precision.HIGHEST)
    offs_ref[...] = offs.astype(jnp.int32)
    cnt = lax.dot_general(newrun, eye, (((0,), (0,)), ((), ())),
                          precision=lax.Precision.HIGHEST)
    cnt_ref[...] = cnt.astype(jnp.int32)


@functools.cache
def _make_sc_scatter():
    mesh = plsc.VectorSubcoreMesh(core_axis_name="c", subcore_axis_name="s")

    @functools.partial(
        pl.kernel,
        mesh=mesh,
        out_type=[
            jax.ShapeDtypeStruct((ITEMS,), jnp.float32),
            jax.ShapeDtypeStruct((ITEMS,), jnp.int32),
        ],
        scratch_types=[
            pltpu.VMEM((1, NEXP), jnp.int32),
            pltpu.VMEM((NEXP,), jnp.int32),
            pltpu.VMEM((2, BT1), jnp.int32),
            pltpu.VMEM((2, BT1), jnp.float32),
            pltpu.VMEM((IPC,), jnp.int32),
            pltpu.VMEM((IPC,), jnp.float32),
            pltpu.VMEM((IPC,), jnp.int32),
            pltpu.VMEM_SHARED((ITEMS,), jnp.float32),
            pltpu.VMEM_SHARED((ITEMS,), jnp.int32),
            pltpu.SemaphoreType.DMA,
            pltpu.SemaphoreType.DMA,
        ],
        compiler_params=pltpu.CompilerParams(needs_layout_passes=False),
    )
    def _sc_scatter(rk_hbm, val_hbm, offs_hbm, outs_hbm, outt_hbm,
                    offs2_v, offs_v, rk_v, valr_v, pos_v, datf_v, dati_v,
                    sh_f, sh_i, sem_a, sem_b):
        cid = lax.axis_index("c")
        sid = lax.axis_index("s")
        h_rk = pltpu.async_copy(rk_hbm.at[sid], rk_v, sem_a)
        h_val = pltpu.async_copy(val_hbm.at[sid], valr_v, sem_b)
        pltpu.sync_copy(offs_hbm, offs2_v)

        for c in range(4):
            offs_v[pl.ds(c * 16, 16)] = offs2_v[0, pl.ds(c * 16, 16)]

        h_rk.wait()
        h_val.wait()

        tbase = sid * BT1
        lane = lax.broadcasted_iota(jnp.int32, (16,), 0)
        for c in range(IPC // 16):
            col = (c % 64) * 16
            rk = rk_v[c // 64, pl.ds(col, 16)]
            key = lax.rem(rk, NEXP)
            rank = lax.div(rk, NEXP)
            off = plsc.load_gather(offs_v, [key])
            pos_v[pl.ds(c * 16, 16)] = off + rank

            @pl.when(cid == 0)
            def _():
                datf_v[pl.ds(c * 16, 16)] = valr_v[c // 64, pl.ds(col, 16)]

            @pl.when(cid == 1)
            def _():
                dati_v[pl.ds(c * 16, 16)] = tbase + col + lane

        @pl.when(cid == 0)
        def _():
            pltpu.sync_copy(datf_v, sh_f.at[pos_v])

        @pl.when(cid == 1)
        def _():
            pltpu.sync_copy(dati_v, sh_i.at[pos_v])

        plsc.subcore_barrier()

        @pl.when((cid == 0) & (sid == 0))
        def _():
            pltpu.sync_copy(sh_f, outs_hbm)

        @pl.when((cid == 1) & (sid == 0))
        def _():
            pltpu.sync_copy(sh_i, outt_hbm)

    return _sc_scatter


def kernel(x, W, scale, per_expert_scale):
    scale2 = scale.reshape(1, HIDDEN)
    pes2 = per_expert_scale.reshape(NEXP, 1)
    utri = _UTRI
    rk, val, offs, cnt = pl.pallas_call(
        _router_body,
        grid=(NB1,),
        in_specs=[
            pl.BlockSpec((BT1, HIDDEN), lambda i: (i, 0)),
            pl.BlockSpec((NEXP, HIDDEN), lambda i: (0, 0)),
            pl.BlockSpec((1, HIDDEN), lambda i: (0, 0)),
            pl.BlockSpec((NEXP, 1), lambda i: (0, 0)),
            pl.BlockSpec((CH, CH), lambda i: (0, 0)),
        ],
        out_specs=[
            pl.BlockSpec((1, 2, BT1), lambda i: (i, 0, 0)),
            pl.BlockSpec((1, 2, BT1), lambda i: (i, 0, 0)),
            pl.BlockSpec((1, NEXP), lambda i: (0, 0)),
            pl.BlockSpec((1, NEXP), lambda i: (0, 0)),
        ],
        out_shape=[
            jax.ShapeDtypeStruct((NB1, 2, BT1), jnp.int32),
            jax.ShapeDtypeStruct((NB1, 2, BT1), jnp.float32),
            jax.ShapeDtypeStruct((1, NEXP), jnp.int32),
            jax.ShapeDtypeStruct((1, NEXP), jnp.int32),
        ],
        scratch_shapes=[pltpu.VMEM((NEXP, 1), jnp.float32)],
        compiler_params=pltpu.CompilerParams(
            dimension_semantics=("arbitrary",)),
    )(x, W, scale2, pes2, utri)

    out_scores, out_tok = _make_sc_scatter()(rk, val, offs)
    return out_scores, out_tok, cnt.reshape(NEXP)
